# Initial kernel scaffold; baseline (speedup 1.0000x reference)
import jax
import jax.numpy as jnp
from jax.experimental import pallas as pl


def kernel(idx, weight):
    raise NotImplementedError("write your pallas kernel here")


# speedup vs baseline: 115.2401x; 115.2401x over previous
import functools

import jax
import jax.numpy as jnp
from jax import lax
from jax.experimental import pallas as pl
from jax.experimental.pallas import tpu as pltpu
from jax.experimental.pallas import tpu_sc as plsc

_NC = 2
_NS = 16
_NW = _NC * _NS
_LANES = 16
_CHUNK = 12800


def _sc_body(idx_hbm, w_hbm, out_hbm, idx_v, vals_v, sem):
    wid = lax.axis_index("s") * _NC + lax.axis_index("c")
    per_w = idx_hbm.shape[0] // _NW
    base = wid * per_w

    def chunk(g, carry):
        off = base + g * _CHUNK
        pltpu.sync_copy(idx_hbm.at[pl.ds(off, _CHUNK)], idx_v)
        pltpu.async_copy(w_hbm.at[idx_v], vals_v, sem).wait()

        def body(i, c):
            x = vals_v[pl.ds(i * _LANES, _LANES)]
            vals_v[pl.ds(i * _LANES, _LANES)] = 1.0 / (1.0 + jnp.exp(-x))
            return c

        lax.fori_loop(0, _CHUNK // _LANES, body, 0)
        pltpu.sync_copy(vals_v, out_hbm.at[pl.ds(off, _CHUNK)])
        return carry

    lax.fori_loop(0, per_w // _CHUNK, chunk, 0)


def kernel(idx, weight):
    n = idx.shape[0]
    assert n % (_NW * _CHUNK) == 0
    flat_idx = idx.reshape(-1)

    run = pl.kernel(
        _sc_body,
        out_type=jax.ShapeDtypeStruct((n,), jnp.float32),
        mesh=plsc.VectorSubcoreMesh(core_axis_name="c", subcore_axis_name="s"),
        scratch_types=[
            pltpu.VMEM((_CHUNK,), jnp.int32),
            pltpu.VMEM((_CHUNK,), jnp.float32),
            pltpu.SemaphoreType.DMA,
        ],
    )
    out = run(flat_idx, weight)
    return out.reshape(idx.shape)


# speedup vs baseline: 399.7339x; 3.4687x over previous
import jax
import jax.numpy as jnp
from jax import lax
from jax.experimental import pallas as pl
from jax.experimental.pallas import tpu as pltpu
from jax.experimental.pallas import tpu_sc as plsc

_NC = 2
_NS = 16
_NW = _NC * _NS
_L = 16

_NUM_V = 1000000
_PAD_V = 1000448
_TPT = _PAD_V // _NS
_CH1 = _TPT // 4
_C2 = 10240


def _sc_body(idx_hbm, w_hbm, out_hbm, sig_sh, wv,
             idx_a, idx_b, vals_a, vals_b, sem_a, sem_b):
    cid = lax.axis_index("c")
    sid = lax.axis_index("s")
    wid = sid * _NC + cid

    tb = sid * _TPT

    def ph1(i, c):
        b = i * (4 * _L)
        for j in range(4):
            s = pl.ds(b + j * _L, _L)
            x = wv[s]
            wv[s] = 1.0 / (1.0 + jnp.exp(-x))
        return c

    n_main = _CH1 // (4 * _L)
    n_tail = (_CH1 - n_main * 4 * _L) // _L
    for k in range(_TPT // _CH1):
        o = tb + k * _CH1
        pltpu.sync_copy(w_hbm.at[pl.ds(o, _CH1)], wv)
        lax.fori_loop(0, n_main, ph1, 0)
        for t in range(n_tail):
            s = pl.ds(n_main * 4 * _L + t * _L, _L)
            x = wv[s]
            wv[s] = 1.0 / (1.0 + jnp.exp(-x))
        pltpu.sync_copy(wv, sig_sh.at[pl.ds(o, _CH1)])
    plsc.subcore_barrier()

    per_w = idx_hbm.shape[0] // _NW
    base = wid * per_w
    nch = per_w // _C2
    bufs = ((idx_a, vals_a, sem_a), (idx_b, vals_b, sem_b))
    pending = None
    for g in range(nch):
        ib, vb, sm = bufs[g % 2]
        off = base + g * _C2
        pltpu.sync_copy(idx_hbm.at[pl.ds(off, _C2)], ib)
        cp = pltpu.async_copy(sig_sh.at[ib], vb, sm)
        if pending is not None:
            pcp, poff, pvb = pending
            pcp.wait()
            pltpu.sync_copy(pvb, out_hbm.at[pl.ds(poff, _C2)])
        pending = (cp, off, vb)
    pcp, poff, pvb = pending
    pcp.wait()
    pltpu.sync_copy(pvb, out_hbm.at[pl.ds(poff, _C2)])


def kernel(idx, weight):
    n = idx.shape[0]
    assert n % (_NW * _C2) == 0
    flat_idx = idx.reshape(-1)
    w_pad = jnp.pad(weight, (0, _PAD_V - weight.shape[0]))

    run = pl.kernel(
        _sc_body,
        out_type=jax.ShapeDtypeStruct((n,), jnp.float32),
        mesh=plsc.VectorSubcoreMesh(core_axis_name="c", subcore_axis_name="s"),
        scratch_types=[
            pltpu.VMEM_SHARED((_PAD_V,), jnp.float32),
            pltpu.VMEM((_CH1,), jnp.float32),
            pltpu.VMEM((_C2,), jnp.int32),
            pltpu.VMEM((_C2,), jnp.int32),
            pltpu.VMEM((_C2,), jnp.float32),
            pltpu.VMEM((_C2,), jnp.float32),
            pltpu.SemaphoreType.DMA,
            pltpu.SemaphoreType.DMA,
        ],
    )
    out = run(flat_idx, w_pad)
    return out.reshape(idx.shape)


# speedup vs baseline: 411.2686x; 1.0289x over previous
import jax
import jax.numpy as jnp
from jax import lax
from jax.experimental import pallas as pl
from jax.experimental.pallas import tpu as pltpu
from jax.experimental.pallas import tpu_sc as plsc

_NC = 2
_NS = 16
_NW = _NC * _NS
_L = 16

_C2 = 12800
_PAD_V = 1024000
_TPT = _PAD_V // _NS


def _sigmoid_chunk(vb):
    def body(i, c):
        b = i * (4 * _L)
        for j in range(4):
            s = pl.ds(b + j * _L, _L)
            x = vb[s]
            vb[s] = 1.0 / (1.0 + jnp.exp(-x))
        return c

    lax.fori_loop(0, _C2 // (4 * _L), body, 0)


def _sc_body(idx_hbm, w_hbm, out_hbm, w_sh,
             idx_a, idx_b, vals_a, vals_b, sem_a, sem_b):
    cid = lax.axis_index("c")
    sid = lax.axis_index("s")
    wid = sid * _NC + cid

    tb = sid * _TPT
    stage_bufs = ((vals_a, sem_a), (vals_b, sem_b))
    outcps = []
    for k in range(_TPT // _C2):
        vb, sm = stage_bufs[k % 2]
        if k >= 2:
            outcps[k - 2].wait()
        o = tb + k * _C2
        pltpu.sync_copy(w_hbm.at[pl.ds(o, _C2)], vb)
        outcps.append(pltpu.async_copy(vb, w_sh.at[pl.ds(o, _C2)], sm))
    for cp in outcps[-2:]:
        cp.wait()
    plsc.subcore_barrier()

    per_w = idx_hbm.shape[0] // _NW
    base = wid * per_w
    nch = per_w // _C2
    bufs = ((idx_a, vals_a, sem_a), (idx_b, vals_b, sem_b))

    pltpu.sync_copy(idx_hbm.at[pl.ds(base, _C2)], idx_a)
    pending = (pltpu.async_copy(w_sh.at[idx_a], vals_a, sem_a), base, vals_a)
    for g in range(1, nch):
        ib, vb, sm = bufs[g % 2]
        off = base + g * _C2
        pltpu.sync_copy(idx_hbm.at[pl.ds(off, _C2)], ib)
        cp = pltpu.async_copy(w_sh.at[ib], vb, sm)
        pcp, poff, pvb = pending
        pcp.wait()
        _sigmoid_chunk(pvb)
        pltpu.sync_copy(pvb, out_hbm.at[pl.ds(poff, _C2)])
        pending = (cp, off, vb)
    pcp, poff, pvb = pending
    pcp.wait()
    _sigmoid_chunk(pvb)
    pltpu.sync_copy(pvb, out_hbm.at[pl.ds(poff, _C2)])


def kernel(idx, weight):
    n = idx.shape[0]
    assert n % (_NW * _C2) == 0
    flat_idx = idx.reshape(-1)
    w_pad = jnp.pad(weight, (0, _PAD_V - weight.shape[0]))

    run = pl.kernel(
        _sc_body,
        out_type=jax.ShapeDtypeStruct((n,), jnp.float32),
        mesh=plsc.VectorSubcoreMesh(core_axis_name="c", subcore_axis_name="s"),
        scratch_types=[
            pltpu.VMEM_SHARED((_PAD_V,), jnp.float32),
            pltpu.VMEM((_C2,), jnp.int32),
            pltpu.VMEM((_C2,), jnp.int32),
            pltpu.VMEM((_C2,), jnp.float32),
            pltpu.VMEM((_C2,), jnp.float32),
            pltpu.SemaphoreType.DMA,
            pltpu.SemaphoreType.DMA,
        ],
    )
    out = run(flat_idx, w_pad)
    return out.reshape(idx.shape)
